# Initial kernel scaffold; baseline (speedup 1.0000x reference)
import jax
import jax.numpy as jnp
from jax.experimental import pallas as pl


def kernel(node_types, edge_index, edge_type, ptr, emb_table, W_msg, b_msg, W_ih, W_hh, b_ih, b_hh, W_gate, b_gate, W_g2g, b_g2g):
    raise NotImplementedError("write your pallas kernel here")


# speedup vs baseline: 15.6175x; 15.6175x over previous
import functools

import jax
import jax.numpy as jnp
from jax import lax
from jax.experimental import pallas as pl
from jax.experimental.pallas import tpu as pltpu
from jax.experimental.pallas import tpu_sc as plsc

NS = 16
EBLK = 512
CHUNK = 128
ZCH = 128
BN = 2000


@functools.lru_cache(maxsize=None)
def _make_sc_pass(N, NP, EP, T, D):
    EPT = EP // NS
    NBLK = EPT // EBLK
    NCH = EBLK // CHUNK
    NV = CHUNK // 16
    RZ = NP // NS
    NZ = RZ // ZCH
    mesh = plsc.VectorSubcoreMesh(core_axis_name="c", subcore_axis_name="s")

    @functools.partial(
        pl.kernel,
        mesh=mesh,
        out_type=jax.ShapeDtypeStruct((2 * NP, D), jnp.float32),
        scratch_types=[
            pltpu.VMEM_SHARED((NP, D), jnp.float32),
            pltpu.VMEM((NZ, ZCH), jnp.int32),
            pltpu.VMEM((ZCH, D), jnp.float32),
            pltpu.VMEM((EBLK,), jnp.int32),
            pltpu.VMEM((EBLK,), jnp.int32),
            pltpu.VMEM((EBLK,), jnp.int32),
            pltpu.VMEM((NCH, CHUNK), jnp.int32),
            pltpu.VMEM((NCH, CHUNK), jnp.int32),
            pltpu.VMEM((CHUNK, D), jnp.float32),
            pltpu.SemaphoreType.DMA,
        ],
    )
    def sc_pass(msgs_hbm, src_hbm, dst_hbm, typ_hbm, zeros_hbm, agg_out,
                acc_sp, lidx, zbuf, sbuf, dbuf, tbuf, gidx, sidx, rows, sem):
        c = lax.axis_index("c")
        s = lax.axis_index("s")
        base_row = s * RZ
        for i in range(NZ):
            for v in range(ZCH // 16):
                lidx[i, pl.ds(v * 16, 16)] = (
                    lax.broadcasted_iota(jnp.int32, (16,), 0)
                    + (base_row + i * ZCH + v * 16))
        pltpu.sync_copy(zeros_hbm, zbuf)
        for i in range(NZ):
            pltpu.sync_copy(zbuf, acc_sp.at[lidx.at[i]])
        plsc.subcore_barrier()

        def blk_body(b, carry):
            base = s * EPT + b * EBLK
            pltpu.sync_copy(src_hbm.at[pl.ds(base, EBLK)], sbuf)
            pltpu.sync_copy(dst_hbm.at[pl.ds(base, EBLK)], dbuf)
            pltpu.sync_copy(typ_hbm.at[pl.ds(base, EBLK)], tbuf)
            for j in range(NCH):
                for v in range(NV):
                    src_sl = pl.ds(j * CHUNK + v * 16, 16)
                    dst_sl = pl.ds(v * 16, 16)
                    gidx[j, dst_sl] = (tbuf[src_sl] * N + sbuf[src_sl]) * 2 + c
                    sidx[j, dst_sl] = dbuf[src_sl]
            for j in range(NCH):
                pltpu.async_copy(msgs_hbm.at[gidx.at[j]], rows, sem).wait()
                pltpu.sync_copy(rows, acc_sp.at[sidx.at[j]], add=True)
            return carry

        lax.fori_loop(0, NBLK, blk_body, 0)
        plsc.subcore_barrier()
        for i in range(NZ):
            pltpu.async_copy(acc_sp.at[lidx.at[i]], zbuf, sem).wait()
            pltpu.sync_copy(zbuf,
                            agg_out.at[pl.ds(c * NP + base_row + i * ZCH, ZCH)])
        plsc.subcore_barrier()

    return sc_pass


def _embed_body(V, nt_ref, emb_ref, h_ref):
    nt = nt_ref[...]
    vio = lax.broadcasted_iota(jnp.int32, (1, V), 1)
    oh = (nt == vio).astype(jnp.float32)
    h_ref[...] = lax.dot(oh, emb_ref[...],
                         preferred_element_type=jnp.float32)


def _msgs_body(T, D, h_ref, wm_ref, bm_ref, out_ref):
    h = h_ref[...]
    for t in range(T):
        for hh in range(2):
            out_ref[t, :, hh, :] = lax.dot(
                h, wm_ref[t, :, pl.ds(hh * D, D)],
                preferred_element_type=jnp.float32,
            ) + bm_ref[:, t, pl.ds(hh * D, D)]


def _gru_body(D, a0_ref, a1_ref, h_ref, wih_ref, whh_ref, bih_ref, bhh_ref,
              hn_ref):
    h = h_ref[...]
    m0 = jnp.maximum(a0_ref[...], 0.0)
    m1 = jnp.maximum(a1_ref[...], 0.0)
    dn = (((1,), (1,)), ((), ()))
    gi = (lax.dot_general(m0, wih_ref[:, pl.ds(0, D)], dn,
                          preferred_element_type=jnp.float32)
          + lax.dot_general(m1, wih_ref[:, pl.ds(D, D)], dn,
                            preferred_element_type=jnp.float32)
          + bih_ref[...])
    gh = lax.dot_general(h, whh_ref[...], dn,
                         preferred_element_type=jnp.float32) + bhh_ref[...]
    r = jax.nn.sigmoid(gi[:, :D] + gh[:, :D])
    z = jax.nn.sigmoid(gi[:, D:2 * D] + gh[:, D:2 * D])
    n = jnp.tanh(gi[:, 2 * D:] + r * gh[:, 2 * D:])
    hn_ref[...] = (1.0 - z) * n + z * h


def _readout_body(h_ref, wg_ref, bg_ref, wgg_ref, bgg_ref, lo_ref, hi_ref,
                  out_ref):
    i = pl.program_id(0)
    h = h_ref[...]
    attn = jax.nn.sigmoid(
        jnp.sum(h * wg_ref[...], axis=1, keepdims=True) + bg_ref[...])
    dn = (((1,), (1,)), ((), ()))
    hg = lax.dot_general(h, wgg_ref[...], dn,
                         preferred_element_type=jnp.float32) + bgg_ref[...]
    nidx = i * BN + lax.broadcasted_iota(jnp.int32, (BN, 1), 0)
    inseg = jnp.logical_and(nidx >= lo_ref[...], nidx < hi_ref[...])
    w = jnp.where(inseg, attn, 0.0)
    part = lax.dot_general(w, hg, (((0,), (0,)), ((), ())),
                           preferred_element_type=jnp.float32)

    @pl.when(i == 0)
    def _():
        out_ref[...] = part

    @pl.when(i > 0)
    def _():
        out_ref[...] = out_ref[...] + part


def _embed_call(node_types, emb_table):
    N = node_types.shape[0]
    V, D = emb_table.shape
    return pl.pallas_call(
        functools.partial(_embed_body, V),
        grid=(N // BN,),
        in_specs=[
            pl.BlockSpec((BN, 1), lambda i: (i, 0)),
            pl.BlockSpec((V, D), lambda i: (0, 0)),
        ],
        out_specs=pl.BlockSpec((BN, D), lambda i: (i, 0)),
        out_shape=jax.ShapeDtypeStruct((N, D), jnp.float32),
    )(node_types.reshape(N, 1), emb_table)


def _msgs_call(h, W_msg, b_msg):
    N, D = h.shape
    T = W_msg.shape[0]
    HM = W_msg.shape[2]
    return pl.pallas_call(
        functools.partial(_msgs_body, T, D),
        grid=(N // BN,),
        in_specs=[
            pl.BlockSpec((BN, D), lambda i: (i, 0)),
            pl.BlockSpec((T, D, HM), lambda i: (0, 0, 0)),
            pl.BlockSpec((1, T, HM), lambda i: (0, 0, 0)),
        ],
        out_specs=pl.BlockSpec((T, BN, 2, D), lambda i: (0, i, 0, 0)),
        out_shape=jax.ShapeDtypeStruct((T, N, 2, D), jnp.float32),
    )(h, W_msg, b_msg.reshape(1, T, HM))


def _gru_call(a0, a1, h, W_ih, W_hh, b_ih, b_hh):
    N, D = h.shape
    return pl.pallas_call(
        functools.partial(_gru_body, D),
        grid=(N // BN,),
        in_specs=[
            pl.BlockSpec((BN, D), lambda i: (i, 0)),
            pl.BlockSpec((BN, D), lambda i: (i, 0)),
            pl.BlockSpec((BN, D), lambda i: (i, 0)),
            pl.BlockSpec((3 * D, 2 * D), lambda i: (0, 0)),
            pl.BlockSpec((3 * D, D), lambda i: (0, 0)),
            pl.BlockSpec((1, 3 * D), lambda i: (0, 0)),
            pl.BlockSpec((1, 3 * D), lambda i: (0, 0)),
        ],
        out_specs=pl.BlockSpec((BN, D), lambda i: (i, 0)),
        out_shape=jax.ShapeDtypeStruct((N, D), jnp.float32),
    )(a0, a1, h, W_ih, W_hh, b_ih.reshape(1, -1), b_hh.reshape(1, -1))


def _readout_call(h, W_gate, b_gate, W_g2g, b_g2g, ptr):
    N, D = h.shape
    G = ptr.shape[0] - 1
    lo = ptr[:G].reshape(1, G)
    hi = ptr[1:].reshape(1, G)
    return pl.pallas_call(
        _readout_body,
        grid=(N // BN,),
        in_specs=[
            pl.BlockSpec((BN, D), lambda i: (i, 0)),
            pl.BlockSpec((1, D), lambda i: (0, 0)),
            pl.BlockSpec((1, 1), lambda i: (0, 0)),
            pl.BlockSpec((D, D), lambda i: (0, 0)),
            pl.BlockSpec((1, D), lambda i: (0, 0)),
            pl.BlockSpec((1, G), lambda i: (0, 0)),
            pl.BlockSpec((1, G), lambda i: (0, 0)),
        ],
        out_specs=pl.BlockSpec((G, D), lambda i: (0, 0)),
        out_shape=jax.ShapeDtypeStruct((G, D), jnp.float32),
    )(h, W_gate.reshape(1, D), b_gate.reshape(1, 1), W_g2g,
      b_g2g.reshape(1, D), lo, hi)


def kernel(node_types, edge_index, edge_type, ptr, emb_table, W_msg, b_msg,
           W_ih, W_hh, b_ih, b_hh, W_gate, b_gate, W_g2g, b_g2g):
    N = node_types.shape[0]
    E = edge_type.shape[0]
    T, D, HM = W_msg.shape
    PASSES = 3

    EP = -(-E // (NS * EBLK)) * (NS * EBLK)
    NPAD = EP - E
    NP = -(-(N + 1) // (NS * ZCH)) * (NS * ZCH)
    NSCR = NP - N
    src = edge_index[0]
    dst = edge_index[1]
    typ = edge_type
    if NPAD:
        pidx = jnp.arange(NPAD, dtype=jnp.int32)
        src = jnp.concatenate([src, pidx % N])
        dst = jnp.concatenate([dst, N + pidx % NSCR])
        typ = jnp.concatenate([typ, jnp.zeros((NPAD,), jnp.int32)])

    zeros = jnp.zeros((ZCH, D), jnp.float32)
    sc_pass = _make_sc_pass(N, NP, EP, T, D)

    h = _embed_call(node_types, emb_table)
    for _ in range(PASSES):
        msgs = _msgs_call(h, W_msg, b_msg)
        agg = sc_pass(msgs.reshape(T * N * 2, D), src, dst, typ, zeros)
        h = _gru_call(agg[:N], agg[NP:NP + N], h, W_ih, W_hh, b_ih, b_hh)
    h_graph = _readout_call(h, W_gate, b_gate, W_g2g, b_g2g, ptr)
    return (h, h_graph)


# speedup vs baseline: 19.8876x; 1.2734x over previous
import functools

import jax
import jax.numpy as jnp
from jax import lax
from jax.experimental import pallas as pl
from jax.experimental.pallas import tpu as pltpu
from jax.experimental.pallas import tpu_sc as plsc

NS = 16
EBLK = 512
CHUNK = 64
ZCH = 64
BN = 2000


@functools.lru_cache(maxsize=None)
def _make_sc_pass(N, NP, EP, T, D):
    EPT = EP // NS
    NBLK = EPT // EBLK
    NCH = EBLK // CHUNK
    NV = CHUNK // 16
    RZ = NP // NS
    NZ = RZ // ZCH
    mesh = plsc.VectorSubcoreMesh(core_axis_name="c", subcore_axis_name="s")

    @functools.partial(
        pl.kernel,
        mesh=mesh,
        out_type=jax.ShapeDtypeStruct((2 * NP, D), jnp.float32),
        scratch_types=[
            pltpu.VMEM_SHARED((NP, D), jnp.float32),
            pltpu.VMEM((NZ, ZCH), jnp.int32),
            pltpu.VMEM((ZCH, D), jnp.float32),
            pltpu.VMEM((EBLK,), jnp.int32),
            pltpu.VMEM((EBLK,), jnp.int32),
            pltpu.VMEM((EBLK,), jnp.int32),
            pltpu.VMEM((NCH, CHUNK), jnp.int32),
            pltpu.VMEM((NCH, CHUNK), jnp.int32),
        ] + [pltpu.VMEM((CHUNK, D), jnp.float32) for _ in range(3)]
          + [pltpu.SemaphoreType.DMA for _ in range(7)],
    )
    def sc_pass(msgs_hbm, src_hbm, dst_hbm, typ_hbm, zeros_hbm, agg_out,
                acc_sp, lidx, zbuf, sbuf, dbuf, tbuf, gidx, sidx, *rest):
        NB = 3
        rows = rest[:NB]
        gsem = rest[NB:2 * NB]
        ssem = rest[2 * NB:3 * NB]
        sem = rest[3 * NB]
        c = lax.axis_index("c")
        s = lax.axis_index("s")
        base_row = s * RZ
        for i in range(NZ):
            for v in range(ZCH // 16):
                lidx[i, pl.ds(v * 16, 16)] = (
                    lax.broadcasted_iota(jnp.int32, (16,), 0)
                    + (base_row + i * ZCH + v * 16))
        pltpu.sync_copy(zeros_hbm, zbuf)
        for i in range(NZ):
            pltpu.sync_copy(zbuf, acc_sp.at[lidx.at[i]])
        plsc.subcore_barrier()

        def blk_body(b, carry):
            base = s * EPT + b * EBLK
            pltpu.sync_copy(src_hbm.at[pl.ds(base, EBLK)], sbuf)
            pltpu.sync_copy(dst_hbm.at[pl.ds(base, EBLK)], dbuf)
            pltpu.sync_copy(typ_hbm.at[pl.ds(base, EBLK)], tbuf)
            for j in range(NCH):
                for v in range(NV):
                    src_sl = pl.ds(j * CHUNK + v * 16, 16)
                    dst_sl = pl.ds(v * 16, 16)
                    gidx[j, dst_sl] = (tbuf[src_sl] * N + sbuf[src_sl]) * 2 + c
                    sidx[j, dst_sl] = dbuf[src_sl]
            gcp = [None] * NCH
            scp = [None] * NCH
            for j in range(min(NB, NCH)):
                gcp[j] = pltpu.async_copy(msgs_hbm.at[gidx.at[j]],
                                          rows[j % NB], gsem[j % NB])
            for j in range(NCH):
                gcp[j].wait()
                scp[j] = pltpu.async_copy(rows[j % NB], acc_sp.at[sidx.at[j]],
                                          ssem[j % NB], add=True)
                nj = j + NB
                if nj < NCH:
                    scp[j].wait()
                    gcp[nj] = pltpu.async_copy(msgs_hbm.at[gidx.at[nj]],
                                               rows[nj % NB], gsem[nj % NB])
            for j in range(max(0, NCH - NB), NCH):
                scp[j].wait()
            return carry

        lax.fori_loop(0, NBLK, blk_body, 0)
        plsc.subcore_barrier()
        for i in range(NZ):
            pltpu.async_copy(acc_sp.at[lidx.at[i]], zbuf, sem).wait()
            pltpu.sync_copy(zbuf,
                            agg_out.at[pl.ds(c * NP + base_row + i * ZCH, ZCH)])
        plsc.subcore_barrier()

    return sc_pass


def _embed_body(V, nt_ref, emb_ref, h_ref):
    nt = nt_ref[...]
    vio = lax.broadcasted_iota(jnp.int32, (1, V), 1)
    oh = (nt == vio).astype(jnp.float32)
    h_ref[...] = lax.dot(oh, emb_ref[...],
                         preferred_element_type=jnp.float32)


def _msgs_body(T, D, h_ref, wm_ref, bm_ref, out_ref):
    h = h_ref[...]
    for t in range(T):
        for hh in range(2):
            out_ref[t, :, hh, :] = lax.dot(
                h, wm_ref[t, :, pl.ds(hh * D, D)],
                preferred_element_type=jnp.float32,
            ) + bm_ref[:, t, pl.ds(hh * D, D)]


def _gru_body(D, a0_ref, a1_ref, h_ref, wih_ref, whh_ref, bih_ref, bhh_ref,
              hn_ref):
    h = h_ref[...]
    m0 = jnp.maximum(a0_ref[...], 0.0)
    m1 = jnp.maximum(a1_ref[...], 0.0)
    dn = (((1,), (1,)), ((), ()))
    gi = (lax.dot_general(m0, wih_ref[:, pl.ds(0, D)], dn,
                          preferred_element_type=jnp.float32)
          + lax.dot_general(m1, wih_ref[:, pl.ds(D, D)], dn,
                            preferred_element_type=jnp.float32)
          + bih_ref[...])
    gh = lax.dot_general(h, whh_ref[...], dn,
                         preferred_element_type=jnp.float32) + bhh_ref[...]
    r = jax.nn.sigmoid(gi[:, :D] + gh[:, :D])
    z = jax.nn.sigmoid(gi[:, D:2 * D] + gh[:, D:2 * D])
    n = jnp.tanh(gi[:, 2 * D:] + r * gh[:, 2 * D:])
    hn_ref[...] = (1.0 - z) * n + z * h


def _readout_body(h_ref, wg_ref, bg_ref, wgg_ref, bgg_ref, lo_ref, hi_ref,
                  out_ref):
    i = pl.program_id(0)
    h = h_ref[...]
    attn = jax.nn.sigmoid(
        jnp.sum(h * wg_ref[...], axis=1, keepdims=True) + bg_ref[...])
    dn = (((1,), (1,)), ((), ()))
    hg = lax.dot_general(h, wgg_ref[...], dn,
                         preferred_element_type=jnp.float32) + bgg_ref[...]
    nidx = i * BN + lax.broadcasted_iota(jnp.int32, (BN, 1), 0)
    inseg = jnp.logical_and(nidx >= lo_ref[...], nidx < hi_ref[...])
    w = jnp.where(inseg, attn, 0.0)
    part = lax.dot_general(w, hg, (((0,), (0,)), ((), ())),
                           preferred_element_type=jnp.float32)

    @pl.when(i == 0)
    def _():
        out_ref[...] = part

    @pl.when(i > 0)
    def _():
        out_ref[...] = out_ref[...] + part


def _embed_call(node_types, emb_table):
    N = node_types.shape[0]
    V, D = emb_table.shape
    return pl.pallas_call(
        functools.partial(_embed_body, V),
        grid=(N // BN,),
        in_specs=[
            pl.BlockSpec((BN, 1), lambda i: (i, 0)),
            pl.BlockSpec((V, D), lambda i: (0, 0)),
        ],
        out_specs=pl.BlockSpec((BN, D), lambda i: (i, 0)),
        out_shape=jax.ShapeDtypeStruct((N, D), jnp.float32),
    )(node_types.reshape(N, 1), emb_table)


def _msgs_call(h, W_msg, b_msg):
    N, D = h.shape
    T = W_msg.shape[0]
    HM = W_msg.shape[2]
    return pl.pallas_call(
        functools.partial(_msgs_body, T, D),
        grid=(N // BN,),
        in_specs=[
            pl.BlockSpec((BN, D), lambda i: (i, 0)),
            pl.BlockSpec((T, D, HM), lambda i: (0, 0, 0)),
            pl.BlockSpec((1, T, HM), lambda i: (0, 0, 0)),
        ],
        out_specs=pl.BlockSpec((T, BN, 2, D), lambda i: (0, i, 0, 0)),
        out_shape=jax.ShapeDtypeStruct((T, N, 2, D), jnp.float32),
    )(h, W_msg, b_msg.reshape(1, T, HM))


def _gru_call(a0, a1, h, W_ih, W_hh, b_ih, b_hh):
    N, D = h.shape
    return pl.pallas_call(
        functools.partial(_gru_body, D),
        grid=(N // BN,),
        in_specs=[
            pl.BlockSpec((BN, D), lambda i: (i, 0)),
            pl.BlockSpec((BN, D), lambda i: (i, 0)),
            pl.BlockSpec((BN, D), lambda i: (i, 0)),
            pl.BlockSpec((3 * D, 2 * D), lambda i: (0, 0)),
            pl.BlockSpec((3 * D, D), lambda i: (0, 0)),
            pl.BlockSpec((1, 3 * D), lambda i: (0, 0)),
            pl.BlockSpec((1, 3 * D), lambda i: (0, 0)),
        ],
        out_specs=pl.BlockSpec((BN, D), lambda i: (i, 0)),
        out_shape=jax.ShapeDtypeStruct((N, D), jnp.float32),
    )(a0, a1, h, W_ih, W_hh, b_ih.reshape(1, -1), b_hh.reshape(1, -1))


def _readout_call(h, W_gate, b_gate, W_g2g, b_g2g, ptr):
    N, D = h.shape
    G = ptr.shape[0] - 1
    lo = ptr[:G].reshape(1, G)
    hi = ptr[1:].reshape(1, G)
    return pl.pallas_call(
        _readout_body,
        grid=(N // BN,),
        in_specs=[
            pl.BlockSpec((BN, D), lambda i: (i, 0)),
            pl.BlockSpec((1, D), lambda i: (0, 0)),
            pl.BlockSpec((1, 1), lambda i: (0, 0)),
            pl.BlockSpec((D, D), lambda i: (0, 0)),
            pl.BlockSpec((1, D), lambda i: (0, 0)),
            pl.BlockSpec((1, G), lambda i: (0, 0)),
            pl.BlockSpec((1, G), lambda i: (0, 0)),
        ],
        out_specs=pl.BlockSpec((G, D), lambda i: (0, 0)),
        out_shape=jax.ShapeDtypeStruct((G, D), jnp.float32),
    )(h, W_gate.reshape(1, D), b_gate.reshape(1, 1), W_g2g,
      b_g2g.reshape(1, D), lo, hi)


def kernel(node_types, edge_index, edge_type, ptr, emb_table, W_msg, b_msg,
           W_ih, W_hh, b_ih, b_hh, W_gate, b_gate, W_g2g, b_g2g):
    N = node_types.shape[0]
    E = edge_type.shape[0]
    T, D, HM = W_msg.shape
    PASSES = 3

    EP = -(-E // (NS * EBLK)) * (NS * EBLK)
    NPAD = EP - E
    NP = -(-(N + 1) // (NS * ZCH)) * (NS * ZCH)
    NSCR = NP - N
    src = edge_index[0]
    dst = edge_index[1]
    typ = edge_type
    if NPAD:
        pidx = jnp.arange(NPAD, dtype=jnp.int32)
        src = jnp.concatenate([src, pidx % N])
        dst = jnp.concatenate([dst, N + pidx % NSCR])
        typ = jnp.concatenate([typ, jnp.zeros((NPAD,), jnp.int32)])

    zeros = jnp.zeros((ZCH, D), jnp.float32)
    sc_pass = _make_sc_pass(N, NP, EP, T, D)

    h = _embed_call(node_types, emb_table)
    for _ in range(PASSES):
        msgs = _msgs_call(h, W_msg, b_msg)
        agg = sc_pass(msgs.reshape(T * N * 2, D), src, dst, typ, zeros)
        h = _gru_call(agg[:N], agg[NP:NP + N], h, W_ih, W_hh, b_ih, b_hh)
    h_graph = _readout_call(h, W_gate, b_gate, W_g2g, b_g2g, ptr)
    return (h, h_graph)


# speedup vs baseline: 22.2000x; 1.1163x over previous
import functools

import jax
import jax.numpy as jnp
from jax import lax
from jax.experimental import pallas as pl
from jax.experimental.pallas import tpu as pltpu
from jax.experimental.pallas import tpu_sc as plsc

NS = 16
EBLK = 512
CHUNK = 64
ZCH = 64
BN = 2000


@functools.lru_cache(maxsize=None)
def _make_sc_pass(N, NP, EP, T, D):
    EPT = EP // NS
    NBLK = EPT // EBLK
    NCH = EBLK // CHUNK
    NV = CHUNK // 16
    RZ = NP // NS
    NZ = RZ // ZCH
    mesh = plsc.VectorSubcoreMesh(core_axis_name="c", subcore_axis_name="s")

    NB = 4

    @functools.partial(
        pl.kernel,
        mesh=mesh,
        out_type=jax.ShapeDtypeStruct((2 * NP, D), jnp.float32),
        scratch_types=[
            pltpu.VMEM_SHARED((NP, D), jnp.float32),
            pltpu.VMEM((NZ, ZCH), jnp.int32),
            pltpu.VMEM((ZCH, D), jnp.float32),
            pltpu.VMEM((1, 3, EBLK), jnp.int32),
            pltpu.VMEM((NCH, CHUNK), jnp.int32),
            pltpu.VMEM((NCH, CHUNK), jnp.int32),
        ] + [pltpu.VMEM((CHUNK, D), jnp.float32) for _ in range(NB)]
          + [pltpu.SemaphoreType.DMA for _ in range(2 * NB + 1)],
    )
    def sc_pass(msgs_hbm, edges_hbm, zeros_hbm, agg_out,
                acc_sp, lidx, zbuf, ebuf, gidx, sidx, *rest):
        rows = rest[:NB]
        gsem = rest[NB:2 * NB]
        ssem = rest[2 * NB:3 * NB]
        sem = rest[3 * NB]
        c = lax.axis_index("c")
        s = lax.axis_index("s")
        base_row = s * RZ
        for i in range(NZ):
            for v in range(ZCH // 16):
                lidx[i, pl.ds(v * 16, 16)] = (
                    lax.broadcasted_iota(jnp.int32, (16,), 0)
                    + (base_row + i * ZCH + v * 16))
        pltpu.sync_copy(zeros_hbm, zbuf)
        zcp = [pltpu.sync_copy(zbuf, acc_sp.at[lidx.at[i]])
               for i in range(NZ)]
        plsc.subcore_barrier()

        def blk_body(b, carry):
            pltpu.sync_copy(edges_hbm.at[pl.ds(s * NBLK + b, 1)], ebuf)
            for j in range(NCH):
                for v in range(NV):
                    src_sl = pl.ds(j * CHUNK + v * 16, 16)
                    dst_sl = pl.ds(v * 16, 16)
                    sv = ebuf[0, 0, src_sl]
                    dv = ebuf[0, 1, src_sl]
                    tv = ebuf[0, 2, src_sl]
                    gidx[j, dst_sl] = (tv * N + sv) * 2 + c
                    sidx[j, dst_sl] = dv
            gcp = [None] * NCH
            scp = [None] * NCH
            for j in range(min(NB, NCH)):
                gcp[j] = pltpu.async_copy(msgs_hbm.at[gidx.at[j]],
                                          rows[j % NB], gsem[j % NB])
            for j in range(NCH):
                gcp[j].wait()
                scp[j] = pltpu.async_copy(rows[j % NB], acc_sp.at[sidx.at[j]],
                                          ssem[j % NB], add=True)
                nj = j + NB
                if nj < NCH:
                    scp[j].wait()
                    gcp[nj] = pltpu.async_copy(msgs_hbm.at[gidx.at[nj]],
                                               rows[nj % NB], gsem[nj % NB])
            for j in range(max(0, NCH - NB), NCH):
                scp[j].wait()
            return carry

        lax.fori_loop(0, NBLK, blk_body, 0)
        plsc.subcore_barrier()
        gcp = [None] * NZ
        wcp = [None] * NZ
        for i in range(min(NB, NZ)):
            gcp[i] = pltpu.async_copy(acc_sp.at[lidx.at[i]],
                                      rows[i % NB], gsem[i % NB])
        for i in range(NZ):
            gcp[i].wait()
            wcp[i] = pltpu.async_copy(
                rows[i % NB],
                agg_out.at[pl.ds(c * NP + base_row + i * ZCH, ZCH)],
                ssem[i % NB])
            ni = i + NB
            if ni < NZ:
                wcp[i].wait()
                gcp[ni] = pltpu.async_copy(acc_sp.at[lidx.at[ni]],
                                           rows[ni % NB], gsem[ni % NB])
        for i in range(max(0, NZ - NB), NZ):
            wcp[i].wait()
        plsc.subcore_barrier()

    return sc_pass


def _embed_body(V, nt_ref, emb_ref, h_ref):
    nt = nt_ref[...]
    vio = lax.broadcasted_iota(jnp.int32, (1, V), 1)
    oh = (nt == vio).astype(jnp.float32)
    h_ref[...] = lax.dot(oh, emb_ref[...],
                         preferred_element_type=jnp.float32)


def _msgs_body(T, D, h_ref, wm_ref, bm_ref, out_ref):
    h = h_ref[...]
    for t in range(T):
        for hh in range(2):
            out_ref[t, :, hh, :] = lax.dot(
                h, wm_ref[t, :, pl.ds(hh * D, D)],
                preferred_element_type=jnp.float32,
            ) + bm_ref[:, t, pl.ds(hh * D, D)]


def _gru_body(D, a0_ref, a1_ref, h_ref, wih_ref, whh_ref, bih_ref, bhh_ref,
              hn_ref):
    h = h_ref[...]
    m0 = jnp.maximum(a0_ref[...], 0.0)
    m1 = jnp.maximum(a1_ref[...], 0.0)
    dn = (((1,), (1,)), ((), ()))
    gi = (lax.dot_general(m0, wih_ref[:, pl.ds(0, D)], dn,
                          preferred_element_type=jnp.float32)
          + lax.dot_general(m1, wih_ref[:, pl.ds(D, D)], dn,
                            preferred_element_type=jnp.float32)
          + bih_ref[...])
    gh = lax.dot_general(h, whh_ref[...], dn,
                         preferred_element_type=jnp.float32) + bhh_ref[...]
    r = jax.nn.sigmoid(gi[:, :D] + gh[:, :D])
    z = jax.nn.sigmoid(gi[:, D:2 * D] + gh[:, D:2 * D])
    n = jnp.tanh(gi[:, 2 * D:] + r * gh[:, 2 * D:])
    hn_ref[...] = (1.0 - z) * n + z * h


def _readout_body(h_ref, wg_ref, bg_ref, wgg_ref, bgg_ref, lo_ref, hi_ref,
                  out_ref):
    i = pl.program_id(0)
    h = h_ref[...]
    attn = jax.nn.sigmoid(
        jnp.sum(h * wg_ref[...], axis=1, keepdims=True) + bg_ref[...])
    dn = (((1,), (1,)), ((), ()))
    hg = lax.dot_general(h, wgg_ref[...], dn,
                         preferred_element_type=jnp.float32) + bgg_ref[...]
    nidx = i * BN + lax.broadcasted_iota(jnp.int32, (BN, 1), 0)
    inseg = jnp.logical_and(nidx >= lo_ref[...], nidx < hi_ref[...])
    w = jnp.where(inseg, attn, 0.0)
    part = lax.dot_general(w, hg, (((0,), (0,)), ((), ())),
                           preferred_element_type=jnp.float32)

    @pl.when(i == 0)
    def _():
        out_ref[...] = part

    @pl.when(i > 0)
    def _():
        out_ref[...] = out_ref[...] + part


def _embed_call(node_types, emb_table):
    N = node_types.shape[0]
    V, D = emb_table.shape
    return pl.pallas_call(
        functools.partial(_embed_body, V),
        grid=(N // BN,),
        in_specs=[
            pl.BlockSpec((BN, 1), lambda i: (i, 0)),
            pl.BlockSpec((V, D), lambda i: (0, 0)),
        ],
        out_specs=pl.BlockSpec((BN, D), lambda i: (i, 0)),
        out_shape=jax.ShapeDtypeStruct((N, D), jnp.float32),
    )(node_types.reshape(N, 1), emb_table)


def _msgs_call(h, W_msg, b_msg):
    N, D = h.shape
    T = W_msg.shape[0]
    HM = W_msg.shape[2]
    return pl.pallas_call(
        functools.partial(_msgs_body, T, D),
        grid=(N // BN,),
        in_specs=[
            pl.BlockSpec((BN, D), lambda i: (i, 0)),
            pl.BlockSpec((T, D, HM), lambda i: (0, 0, 0)),
            pl.BlockSpec((1, T, HM), lambda i: (0, 0, 0)),
        ],
        out_specs=pl.BlockSpec((T, BN, 2, D), lambda i: (0, i, 0, 0)),
        out_shape=jax.ShapeDtypeStruct((T, N, 2, D), jnp.float32),
    )(h, W_msg, b_msg.reshape(1, T, HM))


def _gru_call(a0, a1, h, W_ih, W_hh, b_ih, b_hh):
    N, D = h.shape
    return pl.pallas_call(
        functools.partial(_gru_body, D),
        grid=(N // BN,),
        in_specs=[
            pl.BlockSpec((BN, D), lambda i: (i, 0)),
            pl.BlockSpec((BN, D), lambda i: (i, 0)),
            pl.BlockSpec((BN, D), lambda i: (i, 0)),
            pl.BlockSpec((3 * D, 2 * D), lambda i: (0, 0)),
            pl.BlockSpec((3 * D, D), lambda i: (0, 0)),
            pl.BlockSpec((1, 3 * D), lambda i: (0, 0)),
            pl.BlockSpec((1, 3 * D), lambda i: (0, 0)),
        ],
        out_specs=pl.BlockSpec((BN, D), lambda i: (i, 0)),
        out_shape=jax.ShapeDtypeStruct((N, D), jnp.float32),
    )(a0, a1, h, W_ih, W_hh, b_ih.reshape(1, -1), b_hh.reshape(1, -1))


def _readout_call(h, W_gate, b_gate, W_g2g, b_g2g, ptr):
    N, D = h.shape
    G = ptr.shape[0] - 1
    lo = ptr[:G].reshape(1, G)
    hi = ptr[1:].reshape(1, G)
    return pl.pallas_call(
        _readout_body,
        grid=(N // BN,),
        in_specs=[
            pl.BlockSpec((BN, D), lambda i: (i, 0)),
            pl.BlockSpec((1, D), lambda i: (0, 0)),
            pl.BlockSpec((1, 1), lambda i: (0, 0)),
            pl.BlockSpec((D, D), lambda i: (0, 0)),
            pl.BlockSpec((1, D), lambda i: (0, 0)),
            pl.BlockSpec((1, G), lambda i: (0, 0)),
            pl.BlockSpec((1, G), lambda i: (0, 0)),
        ],
        out_specs=pl.BlockSpec((G, D), lambda i: (0, 0)),
        out_shape=jax.ShapeDtypeStruct((G, D), jnp.float32),
    )(h, W_gate.reshape(1, D), b_gate.reshape(1, 1), W_g2g,
      b_g2g.reshape(1, D), lo, hi)


def kernel(node_types, edge_index, edge_type, ptr, emb_table, W_msg, b_msg,
           W_ih, W_hh, b_ih, b_hh, W_gate, b_gate, W_g2g, b_g2g):
    N = node_types.shape[0]
    E = edge_type.shape[0]
    T, D, HM = W_msg.shape
    PASSES = 3

    EP = -(-E // (NS * EBLK)) * (NS * EBLK)
    NPAD = EP - E
    NP = -(-(N + 1) // (NS * ZCH)) * (NS * ZCH)
    NSCR = NP - N
    src = edge_index[0]
    dst = edge_index[1]
    typ = edge_type
    if NPAD:
        pidx = jnp.arange(NPAD, dtype=jnp.int32)
        src = jnp.concatenate([src, pidx % N])
        dst = jnp.concatenate([dst, N + pidx % NSCR])
        typ = jnp.concatenate([typ, jnp.zeros((NPAD,), jnp.int32)])
    edges = jnp.stack([src, dst, typ], 0).reshape(3, EP // EBLK, EBLK)
    edges = edges.transpose(1, 0, 2)

    zeros = jnp.zeros((ZCH, D), jnp.float32)
    sc_pass = _make_sc_pass(N, NP, EP, T, D)

    h = _embed_call(node_types, emb_table)
    for _ in range(PASSES):
        msgs = _msgs_call(h, W_msg, b_msg)
        agg = sc_pass(msgs.reshape(T * N * 2, D), edges, zeros)
        h = _gru_call(agg[:N], agg[NP:NP + N], h, W_ih, W_hh, b_ih, b_hh)
    h_graph = _readout_call(h, W_gate, b_gate, W_g2g, b_g2g, ptr)
    return (h, h_graph)
